# Initial kernel scaffold; baseline (speedup 1.0000x reference)
import jax
import jax.numpy as jnp
from jax.experimental import pallas as pl


def kernel(input, W1, W2):
    raise NotImplementedError("write your pallas kernel here")


# speedup vs baseline: 8.0259x; 8.0259x over previous
import functools

import jax
import jax.numpy as jnp
from jax import lax
from jax.experimental import pallas as pl
from jax.experimental.pallas import tpu as pltpu
from jax.experimental.pallas import tpu_sc as plsc

VOCAB = 100000
EMBED = 128
BATCH = 4096
SEQ = 200

_info = plsc.get_sparse_core_info()
NC, NS, L = _info.num_cores, _info.num_subcores, _info.num_lanes
NW = NC * NS
ROWS = BATCH * SEQ
RPW = ROWS // NW
CHUNK = 100
NCHUNK = RPW // CHUNK
NBUF = 4


def _body(w1_hbm, idx_hbm, w2_hbm, out_hbm,
          idx_v, w2_v, b0, b1, b2, b3,
          g0, g1, g2, g3, s0, s1, s2, s3):
    bufs = [b0, b1, b2, b3]
    gsems = [g0, g1, g2, g3]
    ssems = [s0, s1, s2, s3]

    wid = lax.axis_index("s") * NC + lax.axis_index("c")
    base = wid * RPW

    pltpu.sync_copy(idx_hbm.at[wid], idx_v)
    pltpu.sync_copy(w2_hbm.at[pl.ds(0, SEQ)], w2_v)

    def gather_desc(g, b):
        return pltpu.make_async_copy(w1_hbm.at[idx_v.at[g]], bufs[b], gsems[b])

    def scatter_desc(g, b):
        return pltpu.make_async_copy(
            bufs[b], out_hbm.at[pl.ds(base + g * CHUNK, CHUNK)], ssems[b])

    gather_desc(0, 0).start()
    gather_desc(1, 1).start()

    @pl.loop(0, NCHUNK, step=NBUF)
    def _chunks(G):
        for b in range(NBUF):
            g = G + b
            gather_desc(g, b).wait()

            s_off = lax.rem(g, 2) * CHUNK
            buf = bufs[b]

            @pl.loop(0, CHUNK)
            def _rows(r):
                for c in range(EMBED // L):
                    sl = pl.ds(c * L, L)
                    buf[r, sl] = buf[r, sl] + w2_v[s_off + r, sl]

            scatter_desc(g, b).start()

            b2 = (b + 2) % NBUF

            @pl.when(g >= 2)
            def _():
                scatter_desc(g - 2, b2).wait()

            @pl.when(g + 2 < NCHUNK)
            def _():
                gather_desc(g + 2, b2).start()

    scatter_desc(NCHUNK - 2, (NCHUNK - 2) % NBUF).wait()
    scatter_desc(NCHUNK - 1, (NCHUNK - 1) % NBUF).wait()


def _make_kernel():
    mesh = plsc.VectorSubcoreMesh(core_axis_name="c", subcore_axis_name="s")
    return pl.kernel(
        _body,
        out_type=jax.ShapeDtypeStruct((ROWS, EMBED), jnp.float32),
        mesh=mesh,
        compiler_params=pltpu.CompilerParams(use_tc_tiling_on_sc=False),
        scratch_types=[
            pltpu.VMEM((NCHUNK, CHUNK), jnp.int32),
            pltpu.VMEM((SEQ, EMBED), jnp.float32),
        ] + [pltpu.VMEM((CHUNK, EMBED), jnp.float32) for _ in range(NBUF)]
          + [pltpu.SemaphoreType.DMA for _ in range(2 * NBUF)],
    )


_kernel_call = _make_kernel()


@jax.jit
def kernel(input, W1, W2):
    idx3 = input.astype(jnp.int32).reshape(NW, NCHUNK, CHUNK)
    out = _kernel_call(W1, idx3, W2)
    return out.reshape(BATCH, SEQ, EMBED)
